# baseline (device time: 105214 ns/iter reference)
import jax
import jax.numpy as jnp
from jax import lax
from jax.experimental import pallas as pl
from jax.experimental.pallas import tpu as pltpu

N_DEV = 4
S = 1024
HQ = 8
DH = 128
D = HQ * DH
BLK = 64
NRES = 4
SCALE = 0.08838834764831843


def kernel(x, Wq, K_ext, V_ext, Wo):
    x2 = x.reshape(S, D).astype(jnp.bfloat16)
    K2 = K_ext.reshape(S, D).astype(jnp.bfloat16)
    V2 = V_ext.reshape(S, D).astype(jnp.bfloat16)

    def body(x_ref, wq_ref, k_ref, v_ref, wo_ref, out_ref,
             kfull, vfull, sAr, sAl, rAl, rAr, sB, rB, lsem):
        my = lax.axis_index("i")
        left = lax.rem(my + N_DEV - 1, N_DEV)
        right = lax.rem(my + 1, N_DEV)

        ck = pltpu.make_async_copy(k_ref, kfull.at[0], lsem.at[0])
        cv = pltpu.make_async_copy(v_ref, vfull.at[0], lsem.at[1])
        ck.start()
        cv.start()

        barrier = pltpu.get_barrier_semaphore()
        for nbr in (left, right):
            pl.semaphore_signal(barrier, inc=1, device_id=(nbr,),
                                device_id_type=pl.DeviceIdType.MESH)
        pl.semaphore_wait(barrier, 2)

        ck.wait()
        cv.wait()

        aKr = pltpu.make_async_remote_copy(
            src_ref=kfull.at[0], dst_ref=kfull.at[1],
            send_sem=sAr.at[0], recv_sem=rAl.at[0],
            device_id=(right,), device_id_type=pl.DeviceIdType.MESH)
        aVr = pltpu.make_async_remote_copy(
            src_ref=vfull.at[0], dst_ref=vfull.at[1],
            send_sem=sAr.at[1], recv_sem=rAl.at[1],
            device_id=(right,), device_id_type=pl.DeviceIdType.MESH)
        aKl = pltpu.make_async_remote_copy(
            src_ref=kfull.at[0], dst_ref=kfull.at[2],
            send_sem=sAl.at[0], recv_sem=rAr.at[0],
            device_id=(left,), device_id_type=pl.DeviceIdType.MESH)
        aVl = pltpu.make_async_remote_copy(
            src_ref=vfull.at[0], dst_ref=vfull.at[2],
            send_sem=sAl.at[1], recv_sem=rAr.at[1],
            device_id=(left,), device_id_type=pl.DeviceIdType.MESH)
        aVr.start()
        aKl.start()
        aKr.start()
        aVl.start()

        qr_b = {}
        accs = [None] * (NRES * HQ)

        def attend(r, hh, blocks, m_pack, l_pack):
            i = r * HQ + hh
            c0 = DH * hh
            qh = qr_b[r][:, c0:c0 + DH]
            kh = jnp.concatenate(
                [kfull[j, BLK * (r + 4 * m):BLK * (r + 4 * m) + BLK,
                       c0:c0 + DH]
                 for j, m in blocks], axis=0)
            vh = jnp.concatenate(
                [vfull[j, BLK * (r + 4 * m):BLK * (r + 4 * m) + BLK,
                       c0:c0 + DH]
                 for j, m in blocks], axis=0)
            sc = lax.dot_general(
                qh, kh, (((1,), (1,)), ((), ())),
                preferred_element_type=jnp.float32) * SCALE
            mc = jnp.max(sc, axis=1, keepdims=True)
            if m_pack is None:
                e = jnp.exp(sc - mc)
                return (mc, jnp.sum(e, axis=1, keepdims=True),
                        jnp.dot(e.astype(jnp.bfloat16), vh,
                                preferred_element_type=jnp.float32))
            m0 = m_pack[:, i:i + 1]
            l0 = l_pack[:, i:i + 1]
            mn = jnp.maximum(m0, mc)
            alpha = jnp.exp(m0 - mn)
            e = jnp.exp(sc - mn)
            return (mn,
                    l0 * alpha + jnp.sum(e, axis=1, keepdims=True),
                    accs[i] * alpha + jnp.dot(
                        e.astype(jnp.bfloat16), vh,
                        preferred_element_type=jnp.float32))

        def run_stage(blocks, m_pack, l_pack):
            ms, ls = [], []
            for r in range(NRES):
                for hh in range(HQ):
                    m, l, acc = attend(r, hh, blocks, m_pack, l_pack)
                    ms.append(m)
                    ls.append(l)
                    accs[r * HQ + hh] = acc
            return (jnp.concatenate(ms, axis=1),
                    jnp.concatenate(ls, axis=1))

        q = jnp.dot(x_ref[...], wq_ref[...].astype(jnp.bfloat16),
                    preferred_element_type=jnp.float32
                    ).astype(jnp.bfloat16)
        wo_b = wo_ref[...].astype(jnp.bfloat16)
        for r in range(NRES):
            qr_b[r] = jnp.concatenate(
                [q[BLK * (r + 4 * m):BLK * (r + 4 * m) + BLK, :]
                 for m in range(4)], axis=0)
        m_pack, l_pack = run_stage([(0, m) for m in range(4)], None, None)

        bV1 = pltpu.make_async_remote_copy(
            src_ref=vfull.at[1, pl.ds(0, 512)],
            dst_ref=vfull.at[3, pl.ds(0, 512)],
            send_sem=sB.at[0], recv_sem=rB.at[0],
            device_id=(right,), device_id_type=pl.DeviceIdType.MESH)
        bV2 = pltpu.make_async_remote_copy(
            src_ref=vfull.at[1, pl.ds(512, 512)],
            dst_ref=vfull.at[3, pl.ds(512, 512)],
            send_sem=sB.at[1], recv_sem=rB.at[1],
            device_id=(right,), device_id_type=pl.DeviceIdType.MESH)
        bK1 = pltpu.make_async_remote_copy(
            src_ref=kfull.at[2, pl.ds(0, 512)],
            dst_ref=kfull.at[3, pl.ds(0, 512)],
            send_sem=sB.at[2], recv_sem=rB.at[2],
            device_id=(left,), device_id_type=pl.DeviceIdType.MESH)
        bK2 = pltpu.make_async_remote_copy(
            src_ref=kfull.at[2, pl.ds(512, 512)],
            dst_ref=kfull.at[3, pl.ds(512, 512)],
            send_sem=sB.at[3], recv_sem=rB.at[3],
            device_id=(left,), device_id_type=pl.DeviceIdType.MESH)

        aVr.wait_recv()
        bV1.start()
        bV2.start()
        aKl.wait_recv()
        bK1.start()
        bK2.start()
        aKr.wait_recv()
        aVl.wait_recv()

        m_pack, l_pack = run_stage(
            [(1, m) for m in range(4)] + [(2, m) for m in range(4)],
            m_pack, l_pack)

        bV1.wait_recv()
        bK1.wait_recv()
        m_pack, l_pack = run_stage([(3, 0), (3, 1)], m_pack, l_pack)

        bV2.wait_recv()
        bK2.wait_recv()

        aKr.wait_send()
        aVr.wait_send()
        aKl.wait_send()
        aVl.wait_send()
        bV1.wait_send()
        bV2.wait_send()
        bK1.wait_send()
        bK2.wait_send()

        for r in range(NRES):
            ctx_heads = []
            for hh in range(HQ):
                _, l, acc = attend(r, hh, [(3, 2), (3, 3)],
                                   m_pack, l_pack)
                ctx_heads.append((acc / l).astype(jnp.bfloat16))
            ctx_r = jnp.concatenate(ctx_heads, axis=1)
            out_r = jnp.dot(ctx_r, wo_b,
                            preferred_element_type=jnp.float32)
            for m in range(4):
                out_ref[BLK * (r + 4 * m):BLK * (r + 4 * m) + BLK, :] = \
                    out_r[BLK * m:BLK * (m + 1), :]

    out2 = pl.pallas_call(
        body,
        out_shape=jax.ShapeDtypeStruct((S, D), jnp.float32),
        in_specs=[
            pl.BlockSpec(memory_space=pltpu.VMEM),
            pl.BlockSpec(memory_space=pltpu.VMEM),
            pl.BlockSpec(memory_space=pl.ANY),
            pl.BlockSpec(memory_space=pl.ANY),
            pl.BlockSpec(memory_space=pltpu.VMEM),
        ],
        out_specs=pl.BlockSpec(memory_space=pltpu.VMEM),
        scratch_shapes=[
            pltpu.VMEM((N_DEV, S, D), jnp.bfloat16),
            pltpu.VMEM((N_DEV, S, D), jnp.bfloat16),
            pltpu.SemaphoreType.DMA((2,)),
            pltpu.SemaphoreType.DMA((2,)),
            pltpu.SemaphoreType.DMA((2,)),
            pltpu.SemaphoreType.DMA((2,)),
            pltpu.SemaphoreType.DMA((4,)),
            pltpu.SemaphoreType.DMA((4,)),
            pltpu.SemaphoreType.DMA((2,)),
        ],
        compiler_params=pltpu.CompilerParams(
            collective_id=0, vmem_limit_bytes=46 * 1024 * 1024),
    )(x2, Wq, K2, V2, Wo)

    return out2.reshape(1, S, D)


# device time: 103784 ns/iter; 1.0138x vs baseline; 1.0138x over previous
import jax
import jax.numpy as jnp
from jax import lax
from jax.experimental import pallas as pl
from jax.experimental.pallas import tpu as pltpu

N_DEV = 4
S = 1024
HQ = 8
DH = 128
D = HQ * DH
BLK = 64
NRES = 4
SCALE = 0.08838834764831843


def kernel(x, Wq, K_ext, V_ext, Wo):
    x2 = x.reshape(S, D).astype(jnp.bfloat16)
    K2 = K_ext.reshape(S, D).astype(jnp.bfloat16)
    V2 = V_ext.reshape(S, D).astype(jnp.bfloat16)

    def body(x_ref, wq_ref, k_ref, v_ref, wo_ref, out_ref,
             kfull, vfull, sAr, sAl, rAl, rAr, sB, rB, lsem):
        my = lax.axis_index("i")
        left = lax.rem(my + N_DEV - 1, N_DEV)
        right = lax.rem(my + 1, N_DEV)

        ck = pltpu.make_async_copy(k_ref, kfull.at[0], lsem.at[0])
        cv = pltpu.make_async_copy(v_ref, vfull.at[0], lsem.at[1])
        ck.start()
        cv.start()

        barrier = pltpu.get_barrier_semaphore()
        for nbr in (left, right):
            pl.semaphore_signal(barrier, inc=1, device_id=(nbr,),
                                device_id_type=pl.DeviceIdType.MESH)
        pl.semaphore_wait(barrier, 2)

        ck.wait()
        cv.wait()

        aKr = pltpu.make_async_remote_copy(
            src_ref=kfull.at[0], dst_ref=kfull.at[1],
            send_sem=sAr.at[0], recv_sem=rAl.at[0],
            device_id=(right,), device_id_type=pl.DeviceIdType.MESH)
        aVr = pltpu.make_async_remote_copy(
            src_ref=vfull.at[0], dst_ref=vfull.at[1],
            send_sem=sAr.at[1], recv_sem=rAl.at[1],
            device_id=(right,), device_id_type=pl.DeviceIdType.MESH)
        aKl = pltpu.make_async_remote_copy(
            src_ref=kfull.at[0], dst_ref=kfull.at[2],
            send_sem=sAl.at[0], recv_sem=rAr.at[0],
            device_id=(left,), device_id_type=pl.DeviceIdType.MESH)
        aVl = pltpu.make_async_remote_copy(
            src_ref=vfull.at[0], dst_ref=vfull.at[2],
            send_sem=sAl.at[1], recv_sem=rAr.at[1],
            device_id=(left,), device_id_type=pl.DeviceIdType.MESH)
        aVr.start()
        aKl.start()
        aKr.start()
        aVl.start()

        qr_b = {}
        accs = [None] * (NRES * HQ)

        def attend(r, hh, slots, m_pack, l_pack):
            i = r * HQ + hh
            c0 = DH * hh
            qh = qr_b[r][:, c0:c0 + DH]
            kh = jnp.concatenate(
                [kfull[j, BLK * (r + 4 * m):BLK * (r + 4 * m) + BLK,
                       c0:c0 + DH]
                 for j in slots for m in range(4)], axis=0)
            vh = jnp.concatenate(
                [vfull[j, BLK * (r + 4 * m):BLK * (r + 4 * m) + BLK,
                       c0:c0 + DH]
                 for j in slots for m in range(4)], axis=0)
            sc = lax.dot_general(
                qh, kh, (((1,), (1,)), ((), ())),
                preferred_element_type=jnp.float32) * SCALE
            mc = jnp.max(sc, axis=1, keepdims=True)
            if m_pack is None:
                e = jnp.exp(sc - mc)
                return (mc, jnp.sum(e, axis=1, keepdims=True),
                        jnp.dot(e.astype(jnp.bfloat16), vh,
                                preferred_element_type=jnp.float32))
            m0 = m_pack[:, i:i + 1]
            l0 = l_pack[:, i:i + 1]
            mn = jnp.maximum(m0, mc)
            alpha = jnp.exp(m0 - mn)
            e = jnp.exp(sc - mn)
            return (mn,
                    l0 * alpha + jnp.sum(e, axis=1, keepdims=True),
                    accs[i] * alpha + jnp.dot(
                        e.astype(jnp.bfloat16), vh,
                        preferred_element_type=jnp.float32))

        def run_stage(slots, m_pack, l_pack):
            ms, ls = [], []
            for r in range(NRES):
                for hh in range(HQ):
                    m, l, acc = attend(r, hh, slots, m_pack, l_pack)
                    ms.append(m)
                    ls.append(l)
                    accs[r * HQ + hh] = acc
            return (jnp.concatenate(ms, axis=1),
                    jnp.concatenate(ls, axis=1))

        q = jnp.dot(x_ref[...], wq_ref[...].astype(jnp.bfloat16),
                    preferred_element_type=jnp.float32
                    ).astype(jnp.bfloat16)
        wo_b = wo_ref[...].astype(jnp.bfloat16)
        for r in range(NRES):
            qr_b[r] = jnp.concatenate(
                [q[BLK * (r + 4 * m):BLK * (r + 4 * m) + BLK, :]
                 for m in range(4)], axis=0)
        m_pack, l_pack = run_stage((0,), None, None)

        bV = pltpu.make_async_remote_copy(
            src_ref=vfull.at[1], dst_ref=vfull.at[3],
            send_sem=sB.at[0], recv_sem=rB.at[0],
            device_id=(right,), device_id_type=pl.DeviceIdType.MESH)
        bK = pltpu.make_async_remote_copy(
            src_ref=kfull.at[2], dst_ref=kfull.at[3],
            send_sem=sB.at[1], recv_sem=rB.at[1],
            device_id=(left,), device_id_type=pl.DeviceIdType.MESH)

        aVr.wait_recv()
        bV.start()
        aKl.wait_recv()
        bK.start()
        aKr.wait_recv()
        aVl.wait_recv()

        m_pack, l_pack = run_stage((1, 2), m_pack, l_pack)

        bV.wait_recv()
        bK.wait_recv()

        aKr.wait_send()
        aVr.wait_send()
        aKl.wait_send()
        aVl.wait_send()
        bV.wait_send()
        bK.wait_send()

        for r in range(NRES):
            ctx_heads = []
            for hh in range(HQ):
                _, l, acc = attend(r, hh, (3,), m_pack, l_pack)
                ctx_heads.append((acc / l).astype(jnp.bfloat16))
            ctx_r = jnp.concatenate(ctx_heads, axis=1)
            out_r = jnp.dot(ctx_r, wo_b,
                            preferred_element_type=jnp.float32)
            for m in range(4):
                out_ref[BLK * (r + 4 * m):BLK * (r + 4 * m) + BLK, :] = \
                    out_r[BLK * m:BLK * (m + 1), :]

    out2 = pl.pallas_call(
        body,
        out_shape=jax.ShapeDtypeStruct((S, D), jnp.float32),
        in_specs=[
            pl.BlockSpec(memory_space=pltpu.VMEM),
            pl.BlockSpec(memory_space=pltpu.VMEM),
            pl.BlockSpec(memory_space=pl.ANY),
            pl.BlockSpec(memory_space=pl.ANY),
            pl.BlockSpec(memory_space=pltpu.VMEM),
        ],
        out_specs=pl.BlockSpec(memory_space=pltpu.VMEM),
        scratch_shapes=[
            pltpu.VMEM((N_DEV, S, D), jnp.bfloat16),
            pltpu.VMEM((N_DEV, S, D), jnp.bfloat16),
            pltpu.SemaphoreType.DMA((2,)),
            pltpu.SemaphoreType.DMA((2,)),
            pltpu.SemaphoreType.DMA((2,)),
            pltpu.SemaphoreType.DMA((2,)),
            pltpu.SemaphoreType.DMA((2,)),
            pltpu.SemaphoreType.DMA((2,)),
            pltpu.SemaphoreType.DMA((2,)),
        ],
        compiler_params=pltpu.CompilerParams(
            collective_id=0, vmem_limit_bytes=46 * 1024 * 1024),
    )(x2, Wq, K2, V2, Wo)

    return out2.reshape(1, S, D)
